# baseline (speedup 1.0000x reference)
import functools

import jax
import jax.numpy as jnp
from jax.experimental import pallas as pl

ALPHA = 0.1
TEMP = 10.0
_PREC = jax.lax.Precision.HIGHEST


def _silu(v):
    return v * jax.nn.sigmoid(v)


def _ln(v, g, b):
    mu = jnp.mean(v, axis=-1, keepdims=True)
    var = jnp.mean((v - mu) * (v - mu), axis=-1, keepdims=True)
    return (v - mu) * jax.lax.rsqrt(var + 1e-5) * g + b


def _dot(a, w):
    return jnp.dot(a, w, preferred_element_type=jnp.float32, precision=_PREC)


def _edge_body(pre, e, xd,
               WeT, wd, W1T, b1, W2T, b2, Wc0T, bc0, c1r,
               mg, mb,
               We0eT, We0mT, be0, We1T, be1, eg, eb, eng, enb,
               m_out, xu_out, en_out):
    ev = e[...]
    xdv = xd[...]
    dist = jnp.sqrt(jnp.sum(xdv * xdv, axis=-1, keepdims=True))
    z = pre[...] + _dot(ev, WeT[...]) + dist * wd[...]
    m = _silu(z)
    m = _ln(m, mg[...], mb[...])
    m = _silu(_dot(m, W1T[...]) + b1[...])
    m = _dot(m, W2T[...]) + b2[...]
    cw = _silu(_dot(m, Wc0T[...]) + bc0[...])
    cw = jnp.sum(cw * c1r[...], axis=-1, keepdims=True)
    cw = jnp.tanh(cw / TEMP)
    xu = cw * xdv / (dist + 1e-8)
    en = _dot(ev, We0eT[...]) + _dot(m, We0mT[...]) + be0[...]
    en = _silu(en)
    en = _ln(en, eg[...], eb[...])
    en = _dot(en, We1T[...]) + be1[...]
    en = _ln(ev + en, eng[...], enb[...])
    m_out[...] = m
    xu_out[...] = xu
    en_out[...] = en


def _edge_call(pre, e, xd, wlist):
    E = pre.shape[0]
    BE = 3200 if E % 3200 == 0 else E
    grid = (E // BE,)
    blk = lambda i: (i, 0)
    full = lambda i: (0, 0)
    in_specs = ([pl.BlockSpec((BE, 64), blk),
                 pl.BlockSpec((BE, 16), blk),
                 pl.BlockSpec((BE, 2), blk)]
                + [pl.BlockSpec(w.shape, full) for w in wlist])
    out_specs = [pl.BlockSpec((BE, 64), blk),
                 pl.BlockSpec((BE, 2), blk),
                 pl.BlockSpec((BE, 16), blk)]
    out_shape = [jax.ShapeDtypeStruct((E, 64), jnp.float32),
                 jax.ShapeDtypeStruct((E, 2), jnp.float32),
                 jax.ShapeDtypeStruct((E, 16), jnp.float32)]
    return pl.pallas_call(_edge_body, grid=grid, in_specs=in_specs,
                          out_specs=out_specs, out_shape=out_shape)(
        pre, e, xd, *wlist)


def _project_body(h, WrT, WcTb0_w, b0, hr_out, hc_out):
    hv = h[...]
    hr_out[...] = _dot(hv, WrT[...])
    hc_out[...] = _dot(hv, WcTb0_w[...]) + b0[...]


def _project_call(h, WrT, WcT, b0):
    N = h.shape[0]
    BN = 5000 if N % 5000 == 0 else N
    grid = (N // BN,)
    blk = lambda i: (i, 0)
    full = lambda i: (0, 0)
    in_specs = [pl.BlockSpec((BN, 64), blk),
                pl.BlockSpec((64, 64), full),
                pl.BlockSpec((64, 64), full),
                pl.BlockSpec((1, 64), full)]
    out_specs = [pl.BlockSpec((BN, 64), blk), pl.BlockSpec((BN, 64), blk)]
    out_shape = [jax.ShapeDtypeStruct((N, 64), jnp.float32)] * 2
    return pl.pallas_call(_project_body, grid=grid, in_specs=in_specs,
                          out_specs=out_specs, out_shape=out_shape)(
        h, WrT, WcT, b0)


def _node_body(h, x, hagg, xagg,
               Wn0hT, Wn0aT, bn0, ng, nb, Wn1T, bn1, nng, nnb,
               h_out, x_out):
    hv = h[...]
    hn = _dot(hv, Wn0hT[...]) + _dot(hagg[...], Wn0aT[...]) + bn0[...]
    hn = _silu(hn)
    hn = _ln(hn, ng[...], nb[...])
    hn = _dot(hn, Wn1T[...]) + bn1[...]
    h_out[...] = _ln(hv + hn, nng[...], nnb[...])
    x_out[...] = x[...] + ALPHA * xagg[...]


def _node_call(h, x, hagg, xagg, wlist):
    N = h.shape[0]
    BN = 5000 if N % 5000 == 0 else N
    grid = (N // BN,)
    blk = lambda i: (i, 0)
    full = lambda i: (0, 0)
    in_specs = ([pl.BlockSpec((BN, 64), blk),
                 pl.BlockSpec((BN, 2), blk),
                 pl.BlockSpec((BN, 64), blk),
                 pl.BlockSpec((BN, 2), blk)]
                + [pl.BlockSpec(w.shape, full) for w in wlist])
    out_specs = [pl.BlockSpec((BN, 64), blk), pl.BlockSpec((BN, 2), blk)]
    out_shape = [jax.ShapeDtypeStruct((N, 64), jnp.float32),
                 jax.ShapeDtypeStruct((N, 2), jnp.float32)]
    return pl.pallas_call(_node_body, grid=grid, in_specs=in_specs,
                          out_specs=out_specs, out_shape=out_shape)(
        h, x, hagg, xagg, *wlist)


def kernel(h, x, e, params, edge_index):
    row = edge_index[0]
    col = edge_index[1]
    N = h.shape[0]
    for p in params:
        W0, b0 = p['m0']
        WrT = W0[:, :64].T
        WcT = W0[:, 64:128].T
        WeT = W0[:, 128:144].T
        wd = W0[:, 144].reshape(1, 64)
        W1, b1 = p['m1']
        W2, b2 = p['m2']
        Wc0, bc0 = p['c0']
        c1r = p['c1']
        mg, mb = p['mln']
        We0, be0 = p['e0']
        We0eT = We0[:, :16].T
        We0mT = We0[:, 16:].T
        We1, be1 = p['e1']
        eg, eb = p['eln']
        eng, enb = p['edge_norm']

        Hr, Hc = _project_call(h, WrT, WcT, b0.reshape(1, 64))
        pre = jnp.take(Hr, row, axis=0) + jnp.take(Hc, col, axis=0)
        xd = x[col] - x[row]

        ewlist = [WeT, wd, W1.T, b1.reshape(1, 64), W2.T, b2.reshape(1, 64),
                  Wc0.T, bc0.reshape(1, 64), c1r,
                  mg.reshape(1, 64), mb.reshape(1, 64),
                  We0eT, We0mT, be0.reshape(1, 64), We1.T, be1.reshape(1, 16),
                  eg.reshape(1, 64), eb.reshape(1, 64),
                  eng.reshape(1, 16), enb.reshape(1, 16)]
        m, xu, e = _edge_call(pre, e, xd, ewlist)

        hagg = jax.ops.segment_sum(m, row, num_segments=N)
        xagg = jax.ops.segment_sum(xu, row, num_segments=N)

        Wn0, bn0 = p['n0']
        nwlist = [Wn0[:, :64].T, Wn0[:, 64:].T, bn0.reshape(1, 64),
                  p['nln'][0].reshape(1, 64), p['nln'][1].reshape(1, 64),
                  p['n1'][0].T, p['n1'][1].reshape(1, 64),
                  p['node_norm'][0].reshape(1, 64),
                  p['node_norm'][1].reshape(1, 64)]
        h, x = _node_call(h, x, hagg, xagg, nwlist)
    return (h, x, e)


# speedup vs baseline: 1.9764x; 1.9764x over previous
import functools

import jax
import jax.numpy as jnp
from jax.experimental import pallas as pl

ALPHA = 0.1
TEMP = 10.0


def _silu(v):
    return v * jax.nn.sigmoid(v)


def _ln(v, g, b):
    mu = jnp.mean(v, axis=-1, keepdims=True)
    var = jnp.mean((v - mu) * (v - mu), axis=-1, keepdims=True)
    return (v - mu) * jax.lax.rsqrt(var + 1e-5) * g + b


def _dot(a, w):
    return jnp.dot(a, w, preferred_element_type=jnp.float32)


def _edge_body(gr, gc, e,
               WeT, W1T, b1, W2T, b2, Wc0T, bc0, c1r,
               mg, mb,
               We0eT, We0mT, be0, We1T, be1, eg, eb, eng, enb,
               scat_out, en_out):
    ev = e[...]
    grv = gr[...]
    gcv = gc[...]
    su = grv + gcv
    di = gcv - grv
    pre = su[:, :64]
    xd = di[:, 64:66]
    dist = jnp.sqrt(jnp.sum(xd * xd, axis=-1, keepdims=True))
    z = pre + _dot(ev, WeT[...]) + dist * bc0[:, 64:]
    m = _silu(z)
    m = _ln(m, mg[...], mb[...])
    m = _silu(_dot(m, W1T[...]) + b1[...])
    m = _dot(m, W2T[...]) + b2[...]
    cw = _silu(_dot(m, Wc0T[...]) + bc0[:, :64])
    cw = jnp.sum(cw * c1r[...], axis=-1, keepdims=True)
    cw = jnp.tanh(cw / TEMP)
    xu = cw * xd / (dist + 1e-8)
    en = _dot(ev, We0eT[...]) + _dot(m, We0mT[...]) + be0[...]
    en = _silu(en)
    en = _ln(en, eg[...], eb[...])
    en = _dot(en, We1T[...]) + be1[...]
    en = _ln(ev + en, eng[...], enb[...])
    scat_out[...] = jnp.concatenate(
        [m, xu, jnp.zeros_like(m[:, :14])], axis=1)
    en_out[...] = en


def _edge_call(gr, gc, e, wlist):
    E = gr.shape[0]
    BE = 3200 if E % 3200 == 0 else E
    grid = (E // BE,)
    blk = lambda i: (i, 0)
    full = lambda i: (0, 0)
    in_specs = ([pl.BlockSpec((BE, 80), blk),
                 pl.BlockSpec((BE, 80), blk),
                 pl.BlockSpec((BE, 16), blk)]
                + [pl.BlockSpec(w.shape, full) for w in wlist])
    out_specs = [pl.BlockSpec((BE, 80), blk),
                 pl.BlockSpec((BE, 16), blk)]
    out_shape = [jax.ShapeDtypeStruct((E, 80), jnp.float32),
                 jax.ShapeDtypeStruct((E, 16), jnp.float32)]
    return pl.pallas_call(_edge_body, grid=grid, in_specs=in_specs,
                          out_specs=out_specs, out_shape=out_shape)(
        gr, gc, e, *wlist)


def _project_body(h, x, WrT, WcT, b0, a_out, b_out):
    hv = h[...]
    xv = x[...]
    pad = jnp.zeros((hv.shape[0], 14), jnp.float32)
    a_out[...] = jnp.concatenate([_dot(hv, WrT[...]), xv, pad], axis=1)
    b_out[...] = jnp.concatenate([_dot(hv, WcT[...]) + b0[...], xv, pad],
                                 axis=1)


def _project_call(h, x, WrT, WcT, b0):
    N = h.shape[0]
    BN = 5000 if N % 5000 == 0 else N
    grid = (N // BN,)
    blk = lambda i: (i, 0)
    full = lambda i: (0, 0)
    in_specs = [pl.BlockSpec((BN, 64), blk),
                pl.BlockSpec((BN, 2), blk),
                pl.BlockSpec((64, 64), full),
                pl.BlockSpec((64, 64), full),
                pl.BlockSpec((1, 64), full)]
    out_specs = [pl.BlockSpec((BN, 80), blk), pl.BlockSpec((BN, 80), blk)]
    out_shape = [jax.ShapeDtypeStruct((N, 80), jnp.float32)] * 2
    return pl.pallas_call(_project_body, grid=grid, in_specs=in_specs,
                          out_specs=out_specs, out_shape=out_shape)(
        h, x, WrT, WcT, b0)


def _node_body(h, x, agg,
               Wn0hT, Wn0aT, bn0, ng, nb, Wn1T, bn1, nng, nnb,
               h_out, x_out):
    hv = h[...]
    aggv = agg[...]
    hn = _dot(hv, Wn0hT[...]) + _dot(aggv[:, :64], Wn0aT[...]) + bn0[...]
    hn = _silu(hn)
    hn = _ln(hn, ng[...], nb[...])
    hn = _dot(hn, Wn1T[...]) + bn1[...]
    h_out[...] = _ln(hv + hn, nng[...], nnb[...])
    x_out[...] = x[...] + ALPHA * aggv[:, 64:66]


def _node_call(h, x, agg, wlist):
    N = h.shape[0]
    BN = 5000 if N % 5000 == 0 else N
    grid = (N // BN,)
    blk = lambda i: (i, 0)
    full = lambda i: (0, 0)
    in_specs = ([pl.BlockSpec((BN, 64), blk),
                 pl.BlockSpec((BN, 2), blk),
                 pl.BlockSpec((BN, 80), blk)]
                + [pl.BlockSpec(w.shape, full) for w in wlist])
    out_specs = [pl.BlockSpec((BN, 64), blk), pl.BlockSpec((BN, 2), blk)]
    out_shape = [jax.ShapeDtypeStruct((N, 64), jnp.float32),
                 jax.ShapeDtypeStruct((N, 2), jnp.float32)]
    return pl.pallas_call(_node_body, grid=grid, in_specs=in_specs,
                          out_specs=out_specs, out_shape=out_shape)(
        h, x, agg, *wlist)


def kernel(h, x, e, params, edge_index):
    row = edge_index[0]
    col = edge_index[1]
    N = h.shape[0]
    for p in params:
        W0, b0 = p['m0']
        WrT = W0[:, :64].T
        WcT = W0[:, 64:128].T
        WeT = W0[:, 128:144].T
        wd = W0[:, 144].reshape(1, 64)
        W1, b1 = p['m1']
        W2, b2 = p['m2']
        Wc0, bc0 = p['c0']
        c1r = p['c1']
        mg, mb = p['mln']
        We0, be0 = p['e0']
        We1, be1 = p['e1']
        eg, eb = p['eln']
        eng, enb = p['edge_norm']

        A, B = _project_call(h, x, WrT, WcT, b0.reshape(1, 64))
        Gr = jnp.take(A, row, axis=0)
        Gc = jnp.take(B, col, axis=0)

        bc0wd = jnp.concatenate([bc0.reshape(1, 64), wd], axis=1)
        ewlist = [WeT, W1.T, b1.reshape(1, 64), W2.T, b2.reshape(1, 64),
                  Wc0.T, bc0wd, c1r,
                  mg.reshape(1, 64), mb.reshape(1, 64),
                  We0[:, :16].T, We0[:, 16:].T, be0.reshape(1, 64),
                  We1.T, be1.reshape(1, 16),
                  eg.reshape(1, 64), eb.reshape(1, 64),
                  eng.reshape(1, 16), enb.reshape(1, 16)]
        scat, e = _edge_call(Gr, Gc, e, ewlist)

        agg = jax.ops.segment_sum(scat, row, num_segments=N)

        Wn0, bn0 = p['n0']
        nwlist = [Wn0[:, :64].T, Wn0[:, 64:].T, bn0.reshape(1, 64),
                  p['nln'][0].reshape(1, 64), p['nln'][1].reshape(1, 64),
                  p['n1'][0].T, p['n1'][1].reshape(1, 64),
                  p['node_norm'][0].reshape(1, 64),
                  p['node_norm'][1].reshape(1, 64)]
        h, x = _node_call(h, x, agg, nwlist)
    return (h, x, e)


# speedup vs baseline: 2.9687x; 1.5021x over previous
import functools

import jax
import jax.numpy as jnp
from jax import lax
from jax.experimental import pallas as pl
from jax.experimental.pallas import tpu as pltpu
from jax.experimental.pallas import tpu_sc as plsc

ALPHA = 0.1
TEMP = 10.0
_NC, _NS = 2, 16
_NW = _NC * _NS


def _sc_gather2(A, B, row, col):
    E = row.shape[0]
    D = A.shape[1]
    per_w = E // _NW
    CH = 40
    GRP = 5
    G = CH * GRP
    n_grp = per_w // G
    mesh = plsc.VectorSubcoreMesh(core_axis_name="c", subcore_axis_name="s",
                                  num_cores=_NC, num_subcores=_NS)

    @functools.partial(
        pl.kernel, mesh=mesh,
        compiler_params=pltpu.CompilerParams(use_tc_tiling_on_sc=False),
        out_type=[jax.ShapeDtypeStruct((E, D), jnp.float32),
                  jax.ShapeDtypeStruct((E, D), jnp.float32)],
        scratch_types=[pltpu.VMEM((G,), jnp.int32),
                       pltpu.VMEM((G,), jnp.int32),
                       pltpu.VMEM((G, D), jnp.float32),
                       pltpu.VMEM((G, D), jnp.float32),
                       pltpu.SemaphoreType.DMA,
                       pltpu.SemaphoreType.DMA],
    )
    def k(a_hbm, b_hbm, row_hbm, col_hbm, gr_hbm, gc_hbm,
          idxr, idxc, bufr, bufc, semr, semc):
        wid = lax.axis_index("s") * _NC + lax.axis_index("c")

        def body(g, _):
            gbase = wid * per_w + g * G
            pltpu.sync_copy(row_hbm.at[pl.ds(gbase, G)], idxr)
            pltpu.sync_copy(col_hbm.at[pl.ds(gbase, G)], idxc)
            ds = []
            for b in range(GRP):
                sl = pl.ds(b * CH, CH)
                ds.append(pltpu.async_copy(a_hbm.at[idxr.at[sl]],
                                           bufr.at[sl], semr))
                ds.append(pltpu.async_copy(b_hbm.at[idxc.at[sl]],
                                           bufc.at[sl], semc))
            for d in ds:
                d.wait()
            pltpu.sync_copy(bufr, gr_hbm.at[pl.ds(gbase, G)])
            pltpu.sync_copy(bufc, gc_hbm.at[pl.ds(gbase, G)])
            return _

        lax.fori_loop(0, n_grp, body, None)

    return k(A, B, row, col)


def _silu(v):
    return v * jax.nn.sigmoid(v)


def _ln(v, g, b):
    mu = jnp.mean(v, axis=-1, keepdims=True)
    var = jnp.mean((v - mu) * (v - mu), axis=-1, keepdims=True)
    return (v - mu) * jax.lax.rsqrt(var + 1e-5) * g + b


def _dot(a, w):
    return jnp.dot(a, w, preferred_element_type=jnp.float32)


def _edge_body(gr, gc, e,
               WeT, W1T, b1, W2T, b2, Wc0T, bc0, c1r,
               mg, mb,
               We0eT, We0mT, be0, We1T, be1, eg, eb, eng, enb,
               scat_out, en_out):
    ev = e[...]
    grv = gr[...]
    gcv = gc[...]
    su = grv + gcv
    di = gcv - grv
    pre = su[:, :64]
    xd = di[:, 64:66]
    dist = jnp.sqrt(jnp.sum(xd * xd, axis=-1, keepdims=True))
    z = pre + _dot(ev, WeT[...]) + dist * bc0[:, 64:]
    m = _silu(z)
    m = _ln(m, mg[...], mb[...])
    m = _silu(_dot(m, W1T[...]) + b1[...])
    m = _dot(m, W2T[...]) + b2[...]
    cw = _silu(_dot(m, Wc0T[...]) + bc0[:, :64])
    cw = jnp.sum(cw * c1r[...], axis=-1, keepdims=True)
    cw = jnp.tanh(cw / TEMP)
    xu = cw * xd / (dist + 1e-8)
    en = _dot(ev, We0eT[...]) + _dot(m, We0mT[...]) + be0[...]
    en = _silu(en)
    en = _ln(en, eg[...], eb[...])
    en = _dot(en, We1T[...]) + be1[...]
    en = _ln(ev + en, eng[...], enb[...])
    scat_out[...] = jnp.concatenate(
        [m, xu, jnp.zeros_like(m[:, :14])], axis=1)
    en_out[...] = en


def _edge_call(gr, gc, e, wlist):
    E = gr.shape[0]
    BE = 3200 if E % 3200 == 0 else E
    grid = (E // BE,)
    blk = lambda i: (i, 0)
    full = lambda i: (0, 0)
    in_specs = ([pl.BlockSpec((BE, 80), blk),
                 pl.BlockSpec((BE, 80), blk),
                 pl.BlockSpec((BE, 16), blk)]
                + [pl.BlockSpec(w.shape, full) for w in wlist])
    out_specs = [pl.BlockSpec((BE, 80), blk),
                 pl.BlockSpec((BE, 16), blk)]
    out_shape = [jax.ShapeDtypeStruct((E, 80), jnp.float32),
                 jax.ShapeDtypeStruct((E, 16), jnp.float32)]
    return pl.pallas_call(_edge_body, grid=grid, in_specs=in_specs,
                          out_specs=out_specs, out_shape=out_shape)(
        gr, gc, e, *wlist)


def _project_body(h, x, WrT, WcT, b0, a_out, b_out):
    hv = h[...]
    xv = x[...]
    pad = jnp.zeros((hv.shape[0], 14), jnp.float32)
    a_out[...] = jnp.concatenate([_dot(hv, WrT[...]), xv, pad], axis=1)
    b_out[...] = jnp.concatenate([_dot(hv, WcT[...]) + b0[...], xv, pad],
                                 axis=1)


def _project_call(h, x, WrT, WcT, b0):
    N = h.shape[0]
    BN = 5000 if N % 5000 == 0 else N
    grid = (N // BN,)
    blk = lambda i: (i, 0)
    full = lambda i: (0, 0)
    in_specs = [pl.BlockSpec((BN, 64), blk),
                pl.BlockSpec((BN, 2), blk),
                pl.BlockSpec((64, 64), full),
                pl.BlockSpec((64, 64), full),
                pl.BlockSpec((1, 64), full)]
    out_specs = [pl.BlockSpec((BN, 80), blk), pl.BlockSpec((BN, 80), blk)]
    out_shape = [jax.ShapeDtypeStruct((N, 80), jnp.float32)] * 2
    return pl.pallas_call(_project_body, grid=grid, in_specs=in_specs,
                          out_specs=out_specs, out_shape=out_shape)(
        h, x, WrT, WcT, b0)


def _node_body(h, x, agg,
               Wn0hT, Wn0aT, bn0, ng, nb, Wn1T, bn1, nng, nnb,
               h_out, x_out):
    hv = h[...]
    aggv = agg[...]
    hn = _dot(hv, Wn0hT[...]) + _dot(aggv[:, :64], Wn0aT[...]) + bn0[...]
    hn = _silu(hn)
    hn = _ln(hn, ng[...], nb[...])
    hn = _dot(hn, Wn1T[...]) + bn1[...]
    h_out[...] = _ln(hv + hn, nng[...], nnb[...])
    x_out[...] = x[...] + ALPHA * aggv[:, 64:66]


def _node_call(h, x, agg, wlist):
    N = h.shape[0]
    BN = 5000 if N % 5000 == 0 else N
    grid = (N // BN,)
    blk = lambda i: (i, 0)
    full = lambda i: (0, 0)
    in_specs = ([pl.BlockSpec((BN, 64), blk),
                 pl.BlockSpec((BN, 2), blk),
                 pl.BlockSpec((BN, 80), blk)]
                + [pl.BlockSpec(w.shape, full) for w in wlist])
    out_specs = [pl.BlockSpec((BN, 64), blk), pl.BlockSpec((BN, 2), blk)]
    out_shape = [jax.ShapeDtypeStruct((N, 64), jnp.float32),
                 jax.ShapeDtypeStruct((N, 2), jnp.float32)]
    return pl.pallas_call(_node_body, grid=grid, in_specs=in_specs,
                          out_specs=out_specs, out_shape=out_shape)(
        h, x, agg, *wlist)


def kernel(h, x, e, params, edge_index):
    row = edge_index[0]
    col = edge_index[1]
    N = h.shape[0]
    for p in params:
        W0, b0 = p['m0']
        WrT = W0[:, :64].T
        WcT = W0[:, 64:128].T
        WeT = W0[:, 128:144].T
        wd = W0[:, 144].reshape(1, 64)
        W1, b1 = p['m1']
        W2, b2 = p['m2']
        Wc0, bc0 = p['c0']
        c1r = p['c1']
        mg, mb = p['mln']
        We0, be0 = p['e0']
        We1, be1 = p['e1']
        eg, eb = p['eln']
        eng, enb = p['edge_norm']

        A, B = _project_call(h, x, WrT, WcT, b0.reshape(1, 64))
        if row.shape[0] % (_NW * 200) == 0:
            Gr, Gc = _sc_gather2(A, B, row, col)
        else:
            Gr = jnp.take(A, row, axis=0)
            Gc = jnp.take(B, col, axis=0)

        bc0wd = jnp.concatenate([bc0.reshape(1, 64), wd], axis=1)
        ewlist = [WeT, W1.T, b1.reshape(1, 64), W2.T, b2.reshape(1, 64),
                  Wc0.T, bc0wd, c1r,
                  mg.reshape(1, 64), mb.reshape(1, 64),
                  We0[:, :16].T, We0[:, 16:].T, be0.reshape(1, 64),
                  We1.T, be1.reshape(1, 16),
                  eg.reshape(1, 64), eb.reshape(1, 64),
                  eng.reshape(1, 16), enb.reshape(1, 16)]
        scat, e = _edge_call(Gr, Gc, e, ewlist)

        agg = jax.ops.segment_sum(scat, row, num_segments=N)

        Wn0, bn0 = p['n0']
        nwlist = [Wn0[:, :64].T, Wn0[:, 64:].T, bn0.reshape(1, 64),
                  p['nln'][0].reshape(1, 64), p['nln'][1].reshape(1, 64),
                  p['n1'][0].T, p['n1'][1].reshape(1, 64),
                  p['node_norm'][0].reshape(1, 64),
                  p['node_norm'][1].reshape(1, 64)]
        h, x = _node_call(h, x, agg, nwlist)
    return (h, x, e)


# speedup vs baseline: 3.4084x; 1.1481x over previous
import functools

import jax
import jax.numpy as jnp
from jax import lax
from jax.experimental import pallas as pl
from jax.experimental.pallas import tpu as pltpu
from jax.experimental.pallas import tpu_sc as plsc

ALPHA = 0.1
TEMP = 10.0
_NC, _NS = 2, 16
_NW = _NC * _NS


def _sc_gather2(A, B, row, col):
    E = row.shape[0]
    D = A.shape[1]
    per_w = E // _NW
    CH = 40
    GRP = 5
    G = CH * GRP
    n_grp = per_w // G
    mesh = plsc.VectorSubcoreMesh(core_axis_name="c", subcore_axis_name="s",
                                  num_cores=_NC, num_subcores=_NS)

    @functools.partial(
        pl.kernel, mesh=mesh,
        out_type=[jax.ShapeDtypeStruct((E, D), jnp.float32),
                  jax.ShapeDtypeStruct((E, D), jnp.float32)],
        scratch_types=[pltpu.VMEM((G,), jnp.int32),
                       pltpu.VMEM((G,), jnp.int32),
                       pltpu.VMEM((G, D), jnp.float32),
                       pltpu.VMEM((G, D), jnp.float32),
                       pltpu.SemaphoreType.DMA,
                       pltpu.SemaphoreType.DMA],
    )
    def k(a_hbm, b_hbm, row_hbm, col_hbm, gr_hbm, gc_hbm,
          idxr, idxc, bufr, bufc, semr, semc):
        wid = lax.axis_index("s") * _NC + lax.axis_index("c")

        def body(g, _):
            gbase = wid * per_w + g * G
            pltpu.sync_copy(row_hbm.at[pl.ds(gbase, G)], idxr)
            pltpu.sync_copy(col_hbm.at[pl.ds(gbase, G)], idxc)
            ds = []
            for b in range(GRP):
                sl = pl.ds(b * CH, CH)
                ds.append(pltpu.async_copy(a_hbm.at[idxr.at[sl]],
                                           bufr.at[sl], semr))
                ds.append(pltpu.async_copy(b_hbm.at[idxc.at[sl]],
                                           bufc.at[sl], semc))
            for d in ds:
                d.wait()
            pltpu.sync_copy(bufr, gr_hbm.at[pl.ds(gbase, G)])
            pltpu.sync_copy(bufc, gc_hbm.at[pl.ds(gbase, G)])
            return _

        lax.fori_loop(0, n_grp, body, None)

    return k(A, B, row, col)


def _silu(v):
    return v * jax.nn.sigmoid(v)


def _ln(v, g, b):
    mu = jnp.mean(v, axis=-1, keepdims=True)
    var = jnp.mean((v - mu) * (v - mu), axis=-1, keepdims=True)
    return (v - mu) * jax.lax.rsqrt(var + 1e-5) * g + b


def _dot(a, w):
    return jnp.dot(a, w, preferred_element_type=jnp.float32)


def _edge_body(gr, gc, e,
               WeT, W1T, b1, W2T, b2, Wc0T, bc0, c1r,
               mg, mb,
               We0eT, We0mT, be0, We1T, be1, eg, eb, eng, enb,
               scat_out, en_out):
    ev = e[...]
    grv = gr[...]
    gcv = gc[...]
    su = grv + gcv
    di = gcv - grv
    pre = su[:, :64]
    xd = di[:, 64:66]
    dist = jnp.sqrt(jnp.sum(xd * xd, axis=-1, keepdims=True))
    z = pre + _dot(ev, WeT[...]) + dist * bc0[:, 64:]
    m = _silu(z)
    m = _ln(m, mg[...], mb[...])
    m = _silu(_dot(m, W1T[...]) + b1[...])
    m = _dot(m, W2T[...]) + b2[...]
    cw = _silu(_dot(m, Wc0T[...]) + bc0[:, :64])
    cw = jnp.sum(cw * c1r[...], axis=-1, keepdims=True)
    cw = jnp.tanh(cw / TEMP)
    xu = cw * xd / (dist + 1e-8)
    en = _dot(ev, We0eT[...]) + _dot(m, We0mT[...]) + be0[...]
    en = _silu(en)
    en = _ln(en, eg[...], eb[...])
    en = _dot(en, We1T[...]) + be1[...]
    en = _ln(ev + en, eng[...], enb[...])
    scat_out[...] = jnp.concatenate(
        [m, xu, jnp.zeros_like(m[:, :14])], axis=1)
    en_out[...] = en


def _edge_call(gr, gc, e, wlist):
    E = gr.shape[0]
    BE = 3200 if E % 3200 == 0 else E
    grid = (E // BE,)
    blk = lambda i: (i, 0)
    full = lambda i: (0, 0)
    in_specs = ([pl.BlockSpec((BE, 128), blk),
                 pl.BlockSpec((BE, 128), blk),
                 pl.BlockSpec((BE, 16), blk)]
                + [pl.BlockSpec(w.shape, full) for w in wlist])
    out_specs = [pl.BlockSpec((BE, 80), blk),
                 pl.BlockSpec((BE, 16), blk)]
    out_shape = [jax.ShapeDtypeStruct((E, 80), jnp.float32),
                 jax.ShapeDtypeStruct((E, 16), jnp.float32)]
    return pl.pallas_call(_edge_body, grid=grid, in_specs=in_specs,
                          out_specs=out_specs, out_shape=out_shape)(
        gr, gc, e, *wlist)


def _project_body(h, x, WrT, WcT, b0, a_out, b_out):
    hv = h[...]
    xv = x[...]
    pad = jnp.zeros((hv.shape[0], 62), jnp.float32)
    a_out[...] = jnp.concatenate([_dot(hv, WrT[...]), xv, pad], axis=1)
    b_out[...] = jnp.concatenate([_dot(hv, WcT[...]) + b0[...], xv, pad],
                                 axis=1)


def _project_call(h, x, WrT, WcT, b0):
    N = h.shape[0]
    BN = 5000 if N % 5000 == 0 else N
    grid = (N // BN,)
    blk = lambda i: (i, 0)
    full = lambda i: (0, 0)
    in_specs = [pl.BlockSpec((BN, 64), blk),
                pl.BlockSpec((BN, 2), blk),
                pl.BlockSpec((64, 64), full),
                pl.BlockSpec((64, 64), full),
                pl.BlockSpec((1, 64), full)]
    out_specs = [pl.BlockSpec((BN, 128), blk), pl.BlockSpec((BN, 128), blk)]
    out_shape = [jax.ShapeDtypeStruct((N, 128), jnp.float32)] * 2
    return pl.pallas_call(_project_body, grid=grid, in_specs=in_specs,
                          out_specs=out_specs, out_shape=out_shape)(
        h, x, WrT, WcT, b0)


def _node_body(h, x, agg,
               Wn0hT, Wn0aT, bn0, ng, nb, Wn1T, bn1, nng, nnb,
               h_out, x_out):
    hv = h[...]
    aggv = agg[...]
    hn = _dot(hv, Wn0hT[...]) + _dot(aggv[:, :64], Wn0aT[...]) + bn0[...]
    hn = _silu(hn)
    hn = _ln(hn, ng[...], nb[...])
    hn = _dot(hn, Wn1T[...]) + bn1[...]
    h_out[...] = _ln(hv + hn, nng[...], nnb[...])
    x_out[...] = x[...] + ALPHA * aggv[:, 64:66]


def _node_call(h, x, agg, wlist):
    N = h.shape[0]
    BN = 5000 if N % 5000 == 0 else N
    grid = (N // BN,)
    blk = lambda i: (i, 0)
    full = lambda i: (0, 0)
    in_specs = ([pl.BlockSpec((BN, 64), blk),
                 pl.BlockSpec((BN, 2), blk),
                 pl.BlockSpec((BN, 80), blk)]
                + [pl.BlockSpec(w.shape, full) for w in wlist])
    out_specs = [pl.BlockSpec((BN, 64), blk), pl.BlockSpec((BN, 2), blk)]
    out_shape = [jax.ShapeDtypeStruct((N, 64), jnp.float32),
                 jax.ShapeDtypeStruct((N, 2), jnp.float32)]
    return pl.pallas_call(_node_body, grid=grid, in_specs=in_specs,
                          out_specs=out_specs, out_shape=out_shape)(
        h, x, agg, *wlist)


def kernel(h, x, e, params, edge_index):
    row = edge_index[0]
    col = edge_index[1]
    N = h.shape[0]
    for p in params:
        W0, b0 = p['m0']
        WrT = W0[:, :64].T
        WcT = W0[:, 64:128].T
        WeT = W0[:, 128:144].T
        wd = W0[:, 144].reshape(1, 64)
        W1, b1 = p['m1']
        W2, b2 = p['m2']
        Wc0, bc0 = p['c0']
        c1r = p['c1']
        mg, mb = p['mln']
        We0, be0 = p['e0']
        We1, be1 = p['e1']
        eg, eb = p['eln']
        eng, enb = p['edge_norm']

        A, B = _project_call(h, x, WrT, WcT, b0.reshape(1, 64))
        if row.shape[0] % (_NW * 200) == 0:
            Gr, Gc = _sc_gather2(A, B, row, col)
        else:
            Gr = jnp.take(A, row, axis=0)
            Gc = jnp.take(B, col, axis=0)

        bc0wd = jnp.concatenate([bc0.reshape(1, 64), wd], axis=1)
        ewlist = [WeT, W1.T, b1.reshape(1, 64), W2.T, b2.reshape(1, 64),
                  Wc0.T, bc0wd, c1r,
                  mg.reshape(1, 64), mb.reshape(1, 64),
                  We0[:, :16].T, We0[:, 16:].T, be0.reshape(1, 64),
                  We1.T, be1.reshape(1, 16),
                  eg.reshape(1, 64), eb.reshape(1, 64),
                  eng.reshape(1, 16), enb.reshape(1, 16)]
        scat, e = _edge_call(Gr, Gc, e, ewlist)

        agg = jax.ops.segment_sum(scat, row, num_segments=N)

        Wn0, bn0 = p['n0']
        nwlist = [Wn0[:, :64].T, Wn0[:, 64:].T, bn0.reshape(1, 64),
                  p['nln'][0].reshape(1, 64), p['nln'][1].reshape(1, 64),
                  p['n1'][0].T, p['n1'][1].reshape(1, 64),
                  p['node_norm'][0].reshape(1, 64),
                  p['node_norm'][1].reshape(1, 64)]
        h, x = _node_call(h, x, agg, nwlist)
    return (h, x, e)


# speedup vs baseline: 3.5143x; 1.0311x over previous
import functools

import jax
import jax.numpy as jnp
from jax import lax
from jax.experimental import pallas as pl
from jax.experimental.pallas import tpu as pltpu
from jax.experimental.pallas import tpu_sc as plsc

ALPHA = 0.1
TEMP = 10.0
_NC, _NS = 2, 16
_NW = _NC * _NS


def _sc_gather2(A, B, row, col):
    E = row.shape[0]
    D = A.shape[1]
    per_w = E // _NW
    CH = 40
    GRP = 5
    G = CH * GRP
    n_grp = per_w // G
    mesh = plsc.VectorSubcoreMesh(core_axis_name="c", subcore_axis_name="s",
                                  num_cores=_NC, num_subcores=_NS)

    @functools.partial(
        pl.kernel, mesh=mesh,
        out_type=[jax.ShapeDtypeStruct((E, D), jnp.float32),
                  jax.ShapeDtypeStruct((E, D), jnp.float32)],
        scratch_types=[pltpu.VMEM((G,), jnp.int32),
                       pltpu.VMEM((G,), jnp.int32),
                       pltpu.VMEM((G, D), jnp.float32),
                       pltpu.VMEM((G, D), jnp.float32),
                       pltpu.SemaphoreType.DMA,
                       pltpu.SemaphoreType.DMA],
    )
    def k(a_hbm, b_hbm, row_hbm, col_hbm, gr_hbm, gc_hbm,
          idxr, idxc, bufr, bufc, semr, semc):
        wid = lax.axis_index("s") * _NC + lax.axis_index("c")

        def body(g, _):
            gbase = wid * per_w + g * G
            pltpu.sync_copy(row_hbm.at[pl.ds(gbase, G)], idxr)
            pltpu.sync_copy(col_hbm.at[pl.ds(gbase, G)], idxc)
            ds = []
            for b in range(GRP):
                sl = pl.ds(b * CH, CH)
                ds.append(pltpu.async_copy(a_hbm.at[idxr.at[sl]],
                                           bufr.at[sl], semr))
                ds.append(pltpu.async_copy(b_hbm.at[idxc.at[sl]],
                                           bufc.at[sl], semc))
            for d in ds:
                d.wait()
            pltpu.sync_copy(bufr, gr_hbm.at[pl.ds(gbase, G)])
            pltpu.sync_copy(bufc, gc_hbm.at[pl.ds(gbase, G)])
            return _

        lax.fori_loop(0, n_grp, body, None)

    return k(A, B, row, col)


def _silu(v):
    return v * jax.nn.sigmoid(v)


def _ln(v, g, b):
    mu = jnp.mean(v, axis=-1, keepdims=True)
    var = jnp.mean((v - mu) * (v - mu), axis=-1, keepdims=True)
    return (v - mu) * jax.lax.rsqrt(var + 1e-5) * g + b


def _dot(a, w):
    return jnp.dot(a, w, preferred_element_type=jnp.float32)


def _edge_body(gr, gc, e,
               WeT, W1T, b1, W2T, b2, Wc0T, bc0, c1r,
               mg, mb,
               We0eT, We0mT, be0, We1T, be1, eg, eb, eng, enb,
               scat_out, en_out):
    ev = e[...]
    grv = gr[...]
    gcv = gc[...]
    su = grv + gcv
    di = gcv - grv
    pre = su[:, :64]
    xd = di[:, 64:66]
    dist = jnp.sqrt(jnp.sum(xd * xd, axis=-1, keepdims=True))
    z = pre + _dot(ev, WeT[...]) + dist * bc0[:, 64:]
    m = _silu(z)
    m = _ln(m, mg[...], mb[...])
    m = _silu(_dot(m, W1T[...]) + b1[...])
    m = _dot(m, W2T[...]) + b2[...]
    cw = _silu(_dot(m, Wc0T[...]) + bc0[:, :64])
    cw = jnp.sum(cw * c1r[...], axis=-1, keepdims=True)
    cw = jnp.tanh(cw / TEMP)
    xu = cw * xd / (dist + 1e-8)
    en = _dot(ev, We0eT[...]) + _dot(m, We0mT[...]) + be0[...]
    en = _silu(en)
    en = _ln(en, eg[...], eb[...])
    en = _dot(en, We1T[...]) + be1[...]
    en = _ln(ev + en, eng[...], enb[...])
    scat_out[...] = jnp.concatenate(
        [m, xu, jnp.zeros_like(m[:, :14])], axis=1)
    en_out[...] = en


def _edge_call(gr, gc, e, wlist):
    E = gr.shape[0]
    BE = 3200 if E % 3200 == 0 else E
    grid = (E // BE,)
    blk = lambda i: (i, 0)
    full = lambda i: (0, 0)
    in_specs = ([pl.BlockSpec((BE, 128), blk),
                 pl.BlockSpec((BE, 128), blk),
                 pl.BlockSpec((BE, 16), blk)]
                + [pl.BlockSpec(w.shape, full) for w in wlist])
    out_specs = [pl.BlockSpec((BE, 80), blk),
                 pl.BlockSpec((BE, 16), blk)]
    out_shape = [jax.ShapeDtypeStruct((E, 80), jnp.float32),
                 jax.ShapeDtypeStruct((E, 16), jnp.float32)]
    return pl.pallas_call(_edge_body, grid=grid, in_specs=in_specs,
                          out_specs=out_specs, out_shape=out_shape)(
        gr, gc, e, *wlist)


def _project_body(h, x, WrT, WcT, b0, a_out, b_out):
    hv = h[...]
    xv = x[...]
    pad = jnp.zeros((hv.shape[0], 62), jnp.float32)
    a_out[...] = jnp.concatenate([_dot(hv, WrT[...]), xv, pad], axis=1)
    b_out[...] = jnp.concatenate([_dot(hv, WcT[...]) + b0[...], xv, pad],
                                 axis=1)


def _project_call(h, x, WrT, WcT, b0):
    N = h.shape[0]
    BN = 5000 if N % 5000 == 0 else N
    grid = (N // BN,)
    blk = lambda i: (i, 0)
    full = lambda i: (0, 0)
    in_specs = [pl.BlockSpec((BN, 64), blk),
                pl.BlockSpec((BN, 2), blk),
                pl.BlockSpec((64, 64), full),
                pl.BlockSpec((64, 64), full),
                pl.BlockSpec((1, 64), full)]
    out_specs = [pl.BlockSpec((BN, 128), blk), pl.BlockSpec((BN, 128), blk)]
    out_shape = [jax.ShapeDtypeStruct((N, 128), jnp.float32)] * 2
    return pl.pallas_call(_project_body, grid=grid, in_specs=in_specs,
                          out_specs=out_specs, out_shape=out_shape)(
        h, x, WrT, WcT, b0)


def _node_body(h, x, agg,
               Wn0hT, Wn0aT, bn0, ng, nb, Wn1T, bn1, nng, nnb,
               h_out, x_out):
    hv = h[...]
    aggv = agg[...]
    hn = _dot(hv, Wn0hT[...]) + _dot(aggv[:, :64], Wn0aT[...]) + bn0[...]
    hn = _silu(hn)
    hn = _ln(hn, ng[...], nb[...])
    hn = _dot(hn, Wn1T[...]) + bn1[...]
    h_out[...] = _ln(hv + hn, nng[...], nnb[...])
    x_out[...] = x[...] + ALPHA * aggv[:, 64:66]


def _node_call(h, x, agg, wlist):
    N = h.shape[0]
    BN = 5000 if N % 5000 == 0 else N
    grid = (N // BN,)
    blk = lambda i: (i, 0)
    full = lambda i: (0, 0)
    in_specs = ([pl.BlockSpec((BN, 64), blk),
                 pl.BlockSpec((BN, 2), blk),
                 pl.BlockSpec((BN, 80), blk)]
                + [pl.BlockSpec(w.shape, full) for w in wlist])
    out_specs = [pl.BlockSpec((BN, 64), blk), pl.BlockSpec((BN, 2), blk)]
    out_shape = [jax.ShapeDtypeStruct((N, 64), jnp.float32),
                 jax.ShapeDtypeStruct((N, 2), jnp.float32)]
    return pl.pallas_call(_node_body, grid=grid, in_specs=in_specs,
                          out_specs=out_specs, out_shape=out_shape)(
        h, x, agg, *wlist)


def kernel(h, x, e, params, edge_index):
    row = edge_index[0]
    col = edge_index[1]
    N = h.shape[0]
    for p in params:
        W0, b0 = p['m0']
        WrT = W0[:, :64].T
        WcT = W0[:, 64:128].T
        WeT = W0[:, 128:144].T
        wd = W0[:, 144].reshape(1, 64)
        W1, b1 = p['m1']
        W2, b2 = p['m2']
        Wc0, bc0 = p['c0']
        c1r = p['c1']
        mg, mb = p['mln']
        We0, be0 = p['e0']
        We1, be1 = p['e1']
        eg, eb = p['eln']
        eng, enb = p['edge_norm']

        A, B = _project_call(h, x, WrT, WcT, b0.reshape(1, 64))
        if row.shape[0] % (_NW * 200) == 0:
            Gr, Gc = _sc_gather2(A, B, row, col)
        else:
            Gr = jnp.take(A, row, axis=0)
            Gc = jnp.take(B, col, axis=0)

        bc0wd = jnp.concatenate([bc0.reshape(1, 64), wd], axis=1)
        ewlist = [WeT, W1.T, b1.reshape(1, 64), W2.T, b2.reshape(1, 64),
                  Wc0.T, bc0wd, c1r,
                  mg.reshape(1, 64), mb.reshape(1, 64),
                  We0[:, :16].T, We0[:, 16:].T, be0.reshape(1, 64),
                  We1.T, be1.reshape(1, 16),
                  eg.reshape(1, 64), eb.reshape(1, 64),
                  eng.reshape(1, 16), enb.reshape(1, 16)]
        E = row.shape[0]
        Eh = (E // 2) // 3200 * 3200
        if 0 < Eh < E:
            sA, eA = _edge_call(Gr[:Eh], Gc[:Eh], e[:Eh], ewlist)
            aggA = jax.ops.segment_sum(sA, row[:Eh], num_segments=N)
            sB, eB = _edge_call(Gr[Eh:], Gc[Eh:], e[Eh:], ewlist)
            aggB = jax.ops.segment_sum(sB, row[Eh:], num_segments=N)
            e = jnp.concatenate([eA, eB], axis=0)
            agg = aggA + aggB
        else:
            scat, e = _edge_call(Gr, Gc, e, ewlist)
            agg = jax.ops.segment_sum(scat, row, num_segments=N)

        Wn0, bn0 = p['n0']
        nwlist = [Wn0[:, :64].T, Wn0[:, 64:].T, bn0.reshape(1, 64),
                  p['nln'][0].reshape(1, 64), p['nln'][1].reshape(1, 64),
                  p['n1'][0].T, p['n1'][1].reshape(1, 64),
                  p['node_norm'][0].reshape(1, 64),
                  p['node_norm'][1].reshape(1, 64)]
        h, x = _node_call(h, x, agg, nwlist)
    return (h, x, e)
